# baseline (speedup 1.0000x reference)
import functools

import numpy as np

import jax
import jax.numpy as jnp
from jax import lax
from jax.experimental import pallas as pl
from jax.experimental.pallas import tpu as pltpu
from jax.experimental.pallas import tpu_sc as plsc

_B, _M, _N, _O, _A, _R = 1, 6890, 64, 64, 6, 5
_MPAD = 6912
_T = _R * 3
_NW = 32
_MBLK = _MPAD // _NW
_CHUNK = 8
_SPJ = _MBLK // _CHUNK
_NCHUNK = _A * _SPJ
_NTRI = _NCHUNK // 3
_GROWS = _R * _MPAD
_BIAS_ROW = _GROWS
_LG = _O // 16
_EC = _CHUNK * _T
_PW = 128
_EW = _MBLK * _PW


def _tc_precompute_body(sig_ref, k_ref, b_ref, out_ref):
    kc = jnp.sum(k_ref[...], axis=0)
    sig = sig_ref[...]
    for r in range(_R):
        out_ref[r * _MPAD:(r + 1) * _MPAD, :] = lax.dot_general(
            sig, kc[r], (((1,), (1,)), ((), ())),
            preferred_element_type=jnp.float32)
    bias = (b_ref.shape[0] * _R) * jnp.sum(b_ref[...], axis=0)
    out_ref[_BIAS_ROW:_BIAS_ROW + 8, :] = jnp.broadcast_to(bias[None, :], (8, _O))


_tc_precompute = pl.pallas_call(
    _tc_precompute_body,
    out_shape=jax.ShapeDtypeStruct((_GROWS + 8, _O), jnp.float32),
)


_SEL_IDX = np.zeros((_A * _R * 3 * 2, _PW), np.float32)
_SEL_W = np.zeros((_A * _R * 3 * 2, _PW), np.float32)
_ROFF = np.zeros((1, _PW), np.float32)
for _a in range(_A):
    for _t in range(_T):
        _SEL_IDX[_a * 30 + 2 * _t, _a * 16 + _t] = 1.0
        _SEL_W[_a * 30 + 2 * _t + 1, _a * 16 + _t] = 1.0
        _ROFF[0, _a * 16 + _t] = (_t // 3) * _MPAD
    _SEL_IDX[_a * 30 + 2 * 14, _a * 16 + 15] = 1.0
    _ROFF[0, _a * 16 + 15] = 4 * _MPAD


def _tc_barprep_body(bar_ref, si_ref, sw_ref, ro_ref, idx_ref, w_ref):
    x = bar_ref[...]
    y = lax.dot_general(x, si_ref[...], (((1,), (0,)), ((), ())),
                        precision=lax.Precision.HIGHEST,
                        preferred_element_type=jnp.float32)
    idx_ref[...] = (y + ro_ref[...]).astype(jnp.int32)
    w_ref[...] = lax.dot_general(x, sw_ref[...], (((1,), (0,)), ((), ())),
                                 precision=lax.Precision.HIGHEST,
                                 preferred_element_type=jnp.float32)


_PBLK = _MPAD // 4


_tc_barprep = pl.pallas_call(
    _tc_barprep_body,
    grid=(4,),
    in_specs=[
        pl.BlockSpec((_PBLK, _A * _R * 3 * 2), lambda i: (i, 0)),
        pl.BlockSpec((_A * _R * 3 * 2, _PW), lambda i: (0, 0)),
        pl.BlockSpec((_A * _R * 3 * 2, _PW), lambda i: (0, 0)),
        pl.BlockSpec((1, _PW), lambda i: (0, 0)),
    ],
    out_specs=[
        pl.BlockSpec((_PBLK, _PW), lambda i: (i, 0)),
        pl.BlockSpec((_PBLK, _PW), lambda i: (i, 0)),
    ],
    out_shape=[
        jax.ShapeDtypeStruct((_MPAD, _PW), jnp.int32),
        jax.ShapeDtypeStruct((_MPAD, _PW), jnp.float32),
    ],
)


_sc_mesh = plsc.VectorSubcoreMesh(core_axis_name="c", subcore_axis_name="s")


@functools.partial(
    pl.kernel,
    out_type=jax.ShapeDtypeStruct((_A * _MPAD // 2, 2 * _O), jnp.float32),
    mesh=_sc_mesh,
    scratch_types=[
        pltpu.VMEM((_EW,), jnp.int32),
        pltpu.VMEM((_EW,), jnp.float32),
        pltpu.VMEM(((_NCHUNK + 3) * _EC + 16,), jnp.int32),
        pltpu.VMEM((_EC, _O), jnp.float32),
        pltpu.VMEM((_EC, _O), jnp.float32),
        pltpu.VMEM((_EC, _O), jnp.float32),
        pltpu.VMEM((_CHUNK // 2, 2 * _O), jnp.float32),
        pltpu.VMEM((_CHUNK // 2, 2 * _O), jnp.float32),
        pltpu.VMEM((_CHUNK // 2, 2 * _O), jnp.float32),
        pltpu.VMEM((1, _O), jnp.float32),
        pltpu.SemaphoreType.DMA,
        pltpu.SemaphoreType.DMA,
        pltpu.SemaphoreType.DMA,
        pltpu.SemaphoreType.DMA,
        pltpu.SemaphoreType.DMA,
        pltpu.SemaphoreType.DMA,
    ],
    compiler_params=pltpu.CompilerParams(use_tc_tiling_on_sc=False),
)
def _sc_gather_combine(g_hbm, fidx_hbm, w_hbm, out_hbm,
                       stg_idx, stg_w, idx_all, rows_a, rows_b, rows_c,
                       out_a, out_b, out_c, bias_v,
                       sem_ga, sem_gb, sem_gc, sem_oa, sem_ob, sem_oc):
    wid = lax.axis_index("s") * 2 + lax.axis_index("c")
    with jax.named_scope("sc_stage"):
        pltpu.sync_copy(g_hbm.at[pl.ds(_BIAS_ROW, 1)], bias_v)
        pltpu.sync_copy(fidx_hbm.at[pl.ds(wid * _EW, _EW)], stg_idx)
        pltpu.sync_copy(w_hbm.at[pl.ds(wid * _EW, _EW)], stg_w)

    with jax.named_scope("sc_reorder"):
        for j in range(_A):
            def reorder(mm, carry, j=j):
                v = stg_idx[pl.ds(mm * _PW + j * 16, 16)]
                idx_all[pl.ds((j * _MBLK + mm) * _T, 16)] = v
                return carry
            lax.fori_loop(0, _MBLK, reorder, 0)
        zero16 = jnp.zeros((16,), jnp.int32)
        for q in range(3 * _CHUNK):
            idx_all[pl.ds(_NCHUNK * _EC + q * _T, 16)] = zero16

    def gather(i, rows_v, sem):
        src = g_hbm.at[idx_all.at[pl.ds(i * _EC, _EC)]]
        return pltpu.async_copy(src, rows_v, sem)

    def gather_wait(rows_v, sem):
        pltpu.make_async_copy(g_hbm.at[idx_all.at[pl.ds(0, _EC)]],
                              rows_v, sem).wait()

    def compute_chunk(i, rows_v, out_v, out_sem, pending):
        j = i // _SPJ
        s = i % _SPJ
        @pl.when(pending)
        def _():
            pltpu.make_async_copy(out_v, out_hbm.at[pl.ds(0, _CHUNK // 2)],
                                  out_sem).wait()
        for c in range(_CHUNK):
            wrow = stg_w[pl.ds((s * _CHUNK + c) * _PW + j * 16, 16)]
            accs = [bias_v[0, pl.ds(l * 16, 16)] for l in range(_LG)]
            for t in range(_T):
                wv = wrow[t]
                for l in range(_LG):
                    accs[l] = accs[l] + wv * rows_v[c * _T + t, pl.ds(l * 16, 16)]
            for l in range(_LG):
                out_v[c // 2, pl.ds((c % 2) * _O + l * 16, 16)] = (
                    jnp.maximum(accs[l], 0.0))
        q0 = (j * _MPAD + wid * _MBLK + s * _CHUNK) // 2
        pltpu.async_copy(out_v, out_hbm.at[pl.ds(q0, _CHUNK // 2)], out_sem)

    gather(0, rows_a, sem_ga)
    gather(1, rows_b, sem_gb)
    gather(2, rows_c, sem_gc)

    bufs = ((rows_a, out_a, sem_ga, sem_oa),
            (rows_b, out_b, sem_gb, sem_ob),
            (rows_c, out_c, sem_gc, sem_oc))

    def tri_body(p, carry):
        i0 = 3 * p
        for d, (rows_v, out_v, gsem, osem) in enumerate(bufs):
            gather_wait(rows_v, gsem)
            compute_chunk(i0 + d, rows_v, out_v, osem, p >= 1)
            gather(i0 + d + 3, rows_v, gsem)
        return carry

    with jax.named_scope("sc_mainloop"):
        lax.fori_loop(0, _NTRI, tri_body, 0)
    for rows_v, out_v, gsem, osem in bufs:
        gather_wait(rows_v, gsem)
        pltpu.make_async_copy(out_v, out_hbm.at[pl.ds(0, _CHUNK // 2)],
                              osem).wait()


def kernel(signal, barycentric, kernels, biases):
    sig = jnp.pad(signal[0], ((0, _MPAD - _M), (0, 0)))
    kern_rs = kernels.reshape(_A * kernels.shape[1], _R, _O, _N)
    g_ext = _tc_precompute(sig, kern_rs, biases)

    bar2 = barycentric.reshape(_M, _A * _R * 3 * 2)
    bar2 = jnp.pad(bar2, ((0, _MPAD - _M), (0, 0)), mode="edge")
    fidx2, w2 = _tc_barprep(bar2, jnp.asarray(_SEL_IDX),
                            jnp.asarray(_SEL_W), jnp.asarray(_ROFF))
    fidx = fidx2.reshape(-1)
    wflat = w2.reshape(-1)

    out = _sc_gather_combine(g_ext, fidx, wflat)
    return out.reshape(_A, _MPAD, _O)[:, :_M, :][None]


# speedup vs baseline: 1.4760x; 1.4760x over previous
import functools

import numpy as np

import jax
import jax.numpy as jnp
from jax import lax
from jax.experimental import pallas as pl
from jax.experimental.pallas import tpu as pltpu
from jax.experimental.pallas import tpu_sc as plsc

_B, _M, _N, _O, _A, _R = 1, 6890, 64, 64, 6, 5
_MPAD = 6912
_T = _R * 3
_NW = 32
_MBLK = _MPAD // _NW
_CHUNK = 8
_SPJ = _MBLK // _CHUNK
_NCHUNK = _A * _SPJ
_NPAIR = _NCHUNK // 2
_GROWS = _R * _MPAD
_BIAS_ROW = _GROWS
_LG = _O // 16
_EC = _CHUNK * _T
_PW = 128
_EW = _MBLK * _PW


def _tc_precompute_body(sig_ref, k_ref, b_ref, out_ref):
    kc = jnp.sum(k_ref[...], axis=0)
    sig = sig_ref[...]
    for r in range(_R):
        out_ref[r * _MPAD:(r + 1) * _MPAD, :] = lax.dot_general(
            sig, kc[r], (((1,), (1,)), ((), ())),
            preferred_element_type=jnp.float32)
    bias = (b_ref.shape[0] * _R) * jnp.sum(b_ref[...], axis=0)
    out_ref[_BIAS_ROW:_BIAS_ROW + 8, :] = jnp.broadcast_to(bias[None, :], (8, _O))


_tc_precompute = pl.pallas_call(
    _tc_precompute_body,
    out_shape=jax.ShapeDtypeStruct((_GROWS + 8, _O), jnp.float32),
)


_SEL_IDX = np.zeros((_A * _R * 3 * 2, _PW), np.float32)
_SEL_W = np.zeros((_A * _R * 3 * 2, _PW), np.float32)
_ROFF = np.zeros((1, _PW), np.float32)
for _a in range(_A):
    for _t in range(_T):
        _SEL_IDX[_a * 30 + 2 * _t, _a * 16 + _t] = 1.0
        _SEL_W[_a * 30 + 2 * _t + 1, _a * 16 + _t] = 1.0
        _ROFF[0, _a * 16 + _t] = (_t // 3) * _MPAD
    _SEL_IDX[_a * 30 + 2 * 14, _a * 16 + 15] = 1.0
    _ROFF[0, _a * 16 + 15] = 4 * _MPAD


def _tc_barprep_body(bar_ref, si_ref, sw_ref, ro_ref, idx_ref, w_ref):
    x = bar_ref[...]
    y = lax.dot_general(x, si_ref[...], (((1,), (0,)), ((), ())),
                        precision=lax.Precision.HIGHEST,
                        preferred_element_type=jnp.float32)
    idx_ref[...] = (y + ro_ref[...]).astype(jnp.int32)
    w_ref[...] = lax.dot_general(x, sw_ref[...], (((1,), (0,)), ((), ())),
                                 precision=lax.Precision.HIGHEST,
                                 preferred_element_type=jnp.float32)


_PBLK = _MPAD // 4


_tc_barprep = pl.pallas_call(
    _tc_barprep_body,
    grid=(4,),
    in_specs=[
        pl.BlockSpec((_PBLK, _A * _R * 3 * 2), lambda i: (i, 0)),
        pl.BlockSpec((_A * _R * 3 * 2, _PW), lambda i: (0, 0)),
        pl.BlockSpec((_A * _R * 3 * 2, _PW), lambda i: (0, 0)),
        pl.BlockSpec((1, _PW), lambda i: (0, 0)),
    ],
    out_specs=[
        pl.BlockSpec((_PBLK, _PW), lambda i: (i, 0)),
        pl.BlockSpec((_PBLK, _PW), lambda i: (i, 0)),
    ],
    out_shape=[
        jax.ShapeDtypeStruct((_MPAD, _PW), jnp.int32),
        jax.ShapeDtypeStruct((_MPAD, _PW), jnp.float32),
    ],
)


_sc_mesh = plsc.VectorSubcoreMesh(core_axis_name="c", subcore_axis_name="s")


@functools.partial(
    pl.kernel,
    out_type=jax.ShapeDtypeStruct((_A * _MPAD // 2, 2 * _O), jnp.float32),
    mesh=_sc_mesh,
    scratch_types=[
        pltpu.VMEM((_EW,), jnp.int32),
        pltpu.VMEM((_EW,), jnp.float32),
        pltpu.VMEM(((_NCHUNK + 1) * _EC + 16,), jnp.int32),
        pltpu.VMEM((_EC, _O), jnp.float32),
        pltpu.VMEM((_EC, _O), jnp.float32),
        pltpu.VMEM((_CHUNK // 2, 2 * _O), jnp.float32),
        pltpu.VMEM((_CHUNK // 2, 2 * _O), jnp.float32),
        pltpu.VMEM((1, _O), jnp.float32),
        pltpu.SemaphoreType.DMA,
        pltpu.SemaphoreType.DMA,
        pltpu.SemaphoreType.DMA,
        pltpu.SemaphoreType.DMA,
    ],
    compiler_params=pltpu.CompilerParams(use_tc_tiling_on_sc=False),
)
def _sc_gather_combine(g_hbm, fidx_hbm, w_hbm, out_hbm,
                       stg_idx, stg_w, idx_all, rows_a, rows_b, out_a, out_b,
                       bias_v, sem_ga, sem_gb, sem_oa, sem_ob):
    wid = lax.axis_index("s") * 2 + lax.axis_index("c")
    with jax.named_scope("sc_stage"):
        pltpu.sync_copy(g_hbm.at[pl.ds(_BIAS_ROW, 1)], bias_v)
        pltpu.sync_copy(fidx_hbm.at[pl.ds(wid * _EW, _EW)], stg_idx)
        pltpu.sync_copy(w_hbm.at[pl.ds(wid * _EW, _EW)], stg_w)

    with jax.named_scope("sc_reorder"):
        for j in range(_A):
            def reorder(mm, carry, j=j):
                v = stg_idx[pl.ds(mm * _PW + j * 16, 16)]
                idx_all[pl.ds((j * _MBLK + mm) * _T, 16)] = v
                return carry
            lax.fori_loop(0, _MBLK, reorder, 0)
        zero16 = jnp.zeros((16,), jnp.int32)
        for q in range(_CHUNK):
            idx_all[pl.ds(_NCHUNK * _EC + q * _T, 16)] = zero16

    def gather(i, rows_v, sem):
        src = g_hbm.at[idx_all.at[pl.ds(i * _EC, _EC)]]
        return pltpu.async_copy(src, rows_v, sem)

    def gather_wait(rows_v, sem):
        pltpu.make_async_copy(g_hbm.at[idx_all.at[pl.ds(0, _EC)]],
                              rows_v, sem).wait()

    def compute_chunk(i, rows_v, out_v, out_sem, pending):
        j = i // _SPJ
        s = i % _SPJ
        @pl.when(pending)
        def _():
            pltpu.make_async_copy(out_v, out_hbm.at[pl.ds(0, _CHUNK // 2)],
                                  out_sem).wait()
        for c in range(_CHUNK):
            wrow = stg_w[pl.ds((s * _CHUNK + c) * _PW + j * 16, 16)]
            accs = [bias_v[0, pl.ds(l * 16, 16)] for l in range(_LG)]
            for t in range(_T):
                wv = wrow[t]
                for l in range(_LG):
                    accs[l] = accs[l] + wv * rows_v[c * _T + t, pl.ds(l * 16, 16)]
            for l in range(_LG):
                out_v[c // 2, pl.ds((c % 2) * _O + l * 16, 16)] = (
                    jnp.maximum(accs[l], 0.0))
        q0 = (j * _MPAD + wid * _MBLK + s * _CHUNK) // 2
        pltpu.async_copy(out_v, out_hbm.at[pl.ds(q0, _CHUNK // 2)], out_sem)

    gather(0, rows_a, sem_ga)

    def pair_body(p, carry):
        i0 = 2 * p
        gather(i0 + 1, rows_b, sem_gb)
        gather_wait(rows_a, sem_ga)
        compute_chunk(i0, rows_a, out_a, sem_oa, p >= 1)
        gather(i0 + 2, rows_a, sem_ga)
        gather_wait(rows_b, sem_gb)
        compute_chunk(i0 + 1, rows_b, out_b, sem_ob, p >= 1)
        return carry

    with jax.named_scope("sc_mainloop"):
        lax.fori_loop(0, _NPAIR, pair_body, 0)
    gather_wait(rows_a, sem_ga)
    pltpu.make_async_copy(out_a, out_hbm.at[pl.ds(0, _CHUNK // 2)], sem_oa).wait()
    pltpu.make_async_copy(out_b, out_hbm.at[pl.ds(0, _CHUNK // 2)], sem_ob).wait()


def kernel(signal, barycentric, kernels, biases):
    sig = jnp.pad(signal[0], ((0, _MPAD - _M), (0, 0)))
    kern_rs = kernels.reshape(_A * kernels.shape[1], _R, _O, _N)
    g_ext = _tc_precompute(sig, kern_rs, biases)

    bar2 = barycentric.reshape(_M, _A * _R * 3 * 2)
    bar2 = jnp.pad(bar2, ((0, _MPAD - _M), (0, 0)), mode="edge")
    fidx2, w2 = _tc_barprep(bar2, jnp.asarray(_SEL_IDX),
                            jnp.asarray(_SEL_W), jnp.asarray(_ROFF))
    fidx = fidx2.reshape(-1)
    wflat = w2.reshape(-1)

    out = _sc_gather_combine(g_ext, fidx, wflat)
    return out.reshape(_A, _MPAD, _O)[:, :_M, :][None]
